# Initial kernel scaffold; baseline (speedup 1.0000x reference)
import jax
import jax.numpy as jnp
from jax.experimental import pallas as pl


def kernel(x, edge_index, edge_weight, W_lift, b_lift, W1, b1, W2, b2, W3, b3, W_out, b_out):
    raise NotImplementedError("write your pallas kernel here")


# speedup vs baseline: 1.0472x; 1.0472x over previous
import functools

import jax
import jax.numpy as jnp
from jax.experimental import pallas as pl
from jax.experimental.pallas import tpu as pltpu

N = 10000
E = 320000
D = 128
H = 100
HP = 128
BM = 2000


def _dense2_body(act, x_ref, w_ref, b_ref, o_ref):
    o_ref[...] = act(
        jnp.dot(x_ref[...], w_ref[...], preferred_element_type=jnp.float32)
        + b_ref[...]
    )


def _dense2(x, wt, b, act):
    m, k = x.shape
    f = wt.shape[1]
    return pl.pallas_call(
        functools.partial(_dense2_body, act),
        grid=(m // BM,),
        in_specs=[
            pl.BlockSpec((BM, k), lambda i: (i, 0)),
            pl.BlockSpec((k, f), lambda i: (0, 0)),
            pl.BlockSpec((1, f), lambda i: (0, 0)),
        ],
        out_specs=pl.BlockSpec((BM, f), lambda i: (i, 0)),
        out_shape=jax.ShapeDtypeStruct((m, f), jnp.float32),
    )(x, wt, b)


def _update_body(h_ref, r_ref, wa_ref, wb_ref, b_ref, o_ref):
    acc = jnp.dot(h_ref[...], wa_ref[...], preferred_element_type=jnp.float32)
    acc += jnp.dot(r_ref[...], wb_ref[...], preferred_element_type=jnp.float32)
    o_ref[...] = jnp.maximum(acc + b_ref[...], 0.0)


def _update(h, r, wat, wbt, b):
    return pl.pallas_call(
        _update_body,
        grid=(N // BM,),
        in_specs=[
            pl.BlockSpec((BM, HP), lambda i: (i, 0)),
            pl.BlockSpec((BM, HP), lambda i: (i, 0)),
            pl.BlockSpec((HP, HP), lambda i: (0, 0)),
            pl.BlockSpec((HP, HP), lambda i: (0, 0)),
            pl.BlockSpec((1, HP), lambda i: (0, 0)),
        ],
        out_specs=pl.BlockSpec((BM, HP), lambda i: (i, 0)),
        out_shape=jax.ShapeDtypeStruct((N, HP), jnp.float32),
    )(h, r, wat, wbt, b)


def _pad_to(a, rows, cols):
    return jnp.pad(a, ((0, rows - a.shape[0]), (0, cols - a.shape[1])))


def kernel(x, edge_index, edge_weight, W_lift, b_lift, W1, b1, W2, b2, W3, b3,
           W_out, b_out):
    src = edge_index[0]
    dst = edge_index[1]
    w = edge_weight[:, 0]

    wl_t = _pad_to(W_lift.T, D, HP)
    bl = jnp.pad(b_lift, (0, HP - H))[None, :]
    mats = []
    for W, b in ((W1, b1), (W2, b2), (W3, b3)):
        wat = _pad_to(W[:, :H].T, HP, HP)
        wbt = _pad_to(W[:, H:].T, HP, HP)
        mats.append((wat, wbt, jnp.pad(b, (0, HP - H))[None, :]))
    wo_t = _pad_to(W_out.T, HP, D)
    bo = b_out[None, :]

    h = _dense2(x, wl_t, bl, jnp.tanh)
    for wat, wbt, b in mats:
        msg = h[src] * w[:, None]
        reduced = jax.ops.segment_sum(msg, dst, num_segments=N)
        h = _update(h, reduced, wat, wbt, b)
    out = _dense2(h, wo_t, bo, jax.nn.sigmoid)
    return out


# speedup vs baseline: 4.0163x; 3.8354x over previous
import functools

import jax
import jax.numpy as jnp
from jax import lax
from jax.experimental import pallas as pl
from jax.experimental.pallas import tpu as pltpu
from jax.experimental.pallas import tpu_sc as plsc

N = 10000
E = 320000
D = 128
H = 100
HP = 128
BM = 2000

NC = 2
NS = 16
NW = NC * NS
EPW = E // NW
C = 80
NCHUNK = EPW // C
NPAD = 10240
RPT = NPAD // NS


def _dense2_body(act, x_ref, w_ref, b_ref, o_ref):
    o_ref[...] = act(
        jnp.dot(x_ref[...], w_ref[...], preferred_element_type=jnp.float32)
        + b_ref[...]
    )


def _dense2(x, wt, b, act):
    m, k = x.shape
    f = wt.shape[1]
    return pl.pallas_call(
        functools.partial(_dense2_body, act),
        grid=(m // BM,),
        in_specs=[
            pl.BlockSpec((BM, k), lambda i: (i, 0)),
            pl.BlockSpec((k, f), lambda i: (0, 0)),
            pl.BlockSpec((1, f), lambda i: (0, 0)),
        ],
        out_specs=pl.BlockSpec((BM, f), lambda i: (i, 0)),
        out_shape=jax.ShapeDtypeStruct((m, f), jnp.float32),
    )(x, wt, b)


def _update_body(h_ref, r_ref, wa_ref, wb_ref, b_ref, o_ref):
    acc = jnp.dot(h_ref[...], wa_ref[...], preferred_element_type=jnp.float32)
    red = r_ref[0] + r_ref[1]
    acc += jnp.dot(red, wb_ref[...], preferred_element_type=jnp.float32)
    o_ref[...] = jnp.maximum(acc + b_ref[...], 0.0)


def _update(h, r, wat, wbt, b):
    return pl.pallas_call(
        _update_body,
        grid=(N // BM,),
        in_specs=[
            pl.BlockSpec((BM, HP), lambda i: (i, 0)),
            pl.BlockSpec((2, BM, HP), lambda i: (0, i, 0)),
            pl.BlockSpec((HP, HP), lambda i: (0, 0)),
            pl.BlockSpec((HP, HP), lambda i: (0, 0)),
            pl.BlockSpec((1, HP), lambda i: (0, 0)),
        ],
        out_specs=pl.BlockSpec((BM, HP), lambda i: (i, 0)),
        out_shape=jax.ShapeDtypeStruct((N, HP), jnp.float32),
    )(h, r, wat, wbt, b)


def _seg_body(h_hbm, edges_hbm, w_hbm, zeros_hbm, out_hbm,
              accum, ebuf, wbuf, rows_v, sem):
    c = lax.axis_index("c")
    s = lax.axis_index("s")
    wid = c * NS + s

    pltpu.sync_copy(zeros_hbm, accum.at[pl.ds(s * RPT, RPT)])
    plsc.subcore_barrier()

    def chunk(k, carry):
        pltpu.sync_copy(edges_hbm.at[wid, k], ebuf)
        pltpu.sync_copy(w_hbm.at[wid, k], wbuf)
        pltpu.async_copy(h_hbm.at[ebuf.at[0]], rows_v, sem).wait()

        def grp(g, carry2):
            w16 = wbuf[0, pl.ds(g * 16, 16)]
            for r in range(16):
                i = g * 16 + r
                wv = jnp.broadcast_to(w16[r], (16,))
                for j in range(HP // 16):
                    sl = pl.ds(j * 16, 16)
                    rows_v[i, sl] = rows_v[i, sl] * wv
            return carry2
        lax.fori_loop(0, C // 16, grp, 0)

        pltpu.sync_copy(rows_v, accum.at[ebuf.at[1]], add=True)
        return carry

    lax.fori_loop(0, NCHUNK, chunk, 0)
    plsc.subcore_barrier()

    pltpu.sync_copy(accum.at[pl.ds(s * RPT, RPT)],
                    out_hbm.at[c, pl.ds(s * RPT, RPT)])


_seg = functools.partial(
    pl.kernel,
    out_type=jax.ShapeDtypeStruct((NC, NPAD, HP), jnp.float32),
    mesh=plsc.VectorSubcoreMesh(core_axis_name="c", subcore_axis_name="s"),
    scratch_types=[
        pltpu.VMEM_SHARED((NPAD, HP), jnp.float32),
        pltpu.VMEM((2, C), jnp.int32),
        pltpu.VMEM((1, C), jnp.float32),
        pltpu.VMEM((C, HP), jnp.float32),
        pltpu.SemaphoreType.DMA,
    ],
)(_seg_body)


def _pad_to(a, rows, cols):
    return jnp.pad(a, ((0, rows - a.shape[0]), (0, cols - a.shape[1])))


def kernel(x, edge_index, edge_weight, W_lift, b_lift, W1, b1, W2, b2, W3, b3,
           W_out, b_out):
    src_t = edge_index[0].reshape(NW, NCHUNK, 1, C)
    dst_t = edge_index[1].reshape(NW, NCHUNK, 1, C)
    edges = jnp.concatenate([src_t, dst_t], axis=2)
    w_t = edge_weight.reshape(NW, NCHUNK, 1, C)
    zeros = jnp.zeros((RPT, HP), jnp.float32)

    wl_t = _pad_to(W_lift.T, D, HP)
    bl = jnp.pad(b_lift, (0, HP - H))[None, :]
    mats = []
    for W, b in ((W1, b1), (W2, b2), (W3, b3)):
        wat = _pad_to(W[:, :H].T, HP, HP)
        wbt = _pad_to(W[:, H:].T, HP, HP)
        mats.append((wat, wbt, jnp.pad(b, (0, HP - H))[None, :]))
    wo_t = _pad_to(W_out.T, HP, D)
    bo = b_out[None, :]

    h = _dense2(x, wl_t, bl, jnp.tanh)
    for wat, wbt, b in mats:
        r = _seg(h, edges, w_t, zeros)
        h = _update(h, r, wat, wbt, b)
    out = _dense2(h, wo_t, bo, jax.nn.sigmoid)
    return out


# speedup vs baseline: 6.8543x; 1.7066x over previous
import functools

import jax
import jax.numpy as jnp
from jax import lax
from jax.experimental import pallas as pl
from jax.experimental.pallas import tpu as pltpu
from jax.experimental.pallas import tpu_sc as plsc

N = 10000
E = 320000
D = 128
H = 100
HP = 128
BM = 2000

NC = 2
NS = 16
NW = NC * NS
EPW = E // NW
C = 80
NCHUNK = EPW // C
NPAD = 10240
RPT = NPAD // NS


def _dense2_body(act, x_ref, w_ref, b_ref, o_ref):
    o_ref[...] = act(
        jnp.dot(x_ref[...], w_ref[...], preferred_element_type=jnp.float32)
        + b_ref[...]
    )


def _dense2(x, wt, b, act):
    m, k = x.shape
    f = wt.shape[1]
    return pl.pallas_call(
        functools.partial(_dense2_body, act),
        grid=(m // BM,),
        in_specs=[
            pl.BlockSpec((BM, k), lambda i: (i, 0)),
            pl.BlockSpec((k, f), lambda i: (0, 0)),
            pl.BlockSpec((1, f), lambda i: (0, 0)),
        ],
        out_specs=pl.BlockSpec((BM, f), lambda i: (i, 0)),
        out_shape=jax.ShapeDtypeStruct((m, f), jnp.float32),
    )(x, wt, b)


def _update_body(h_ref, r_ref, wa_ref, wb_ref, b_ref, o_ref):
    acc = jnp.dot(h_ref[...], wa_ref[...], preferred_element_type=jnp.float32)
    red = r_ref[0] + r_ref[1]
    acc += jnp.dot(red, wb_ref[...], preferred_element_type=jnp.float32)
    o_ref[...] = jnp.maximum(acc + b_ref[...], 0.0)


def _update(h, r, wat, wbt, b):
    return pl.pallas_call(
        _update_body,
        grid=(N // BM,),
        in_specs=[
            pl.BlockSpec((BM, HP), lambda i: (i, 0)),
            pl.BlockSpec((2, BM, HP), lambda i: (0, i, 0)),
            pl.BlockSpec((HP, HP), lambda i: (0, 0)),
            pl.BlockSpec((HP, HP), lambda i: (0, 0)),
            pl.BlockSpec((1, HP), lambda i: (0, 0)),
        ],
        out_specs=pl.BlockSpec((BM, HP), lambda i: (i, 0)),
        out_shape=jax.ShapeDtypeStruct((N, HP), jnp.float32),
    )(h, r, wat, wbt, b)


def _seg_body(h_hbm, src_hbm, dst_hbm, w_hbm, zeros_hbm, out_hbm,
              accum, sb0, sb1, db0, db1, wb0, wb1, rv0, rv1,
              ss0, ss1, sd0, sd1, sw0, sw1, sg0, sg1, sc0, sc1):
    c = lax.axis_index("c")
    s = lax.axis_index("s")
    wid = c * NS + s

    pltpu.sync_copy(zeros_hbm, accum.at[pl.ds(s * RPT, RPT)])
    plsc.subcore_barrier()

    sbuf = (sb0, sb1)
    dbuf = (db0, db1)
    wbuf = (wb0, wb1)
    rows = (rv0, rv1)
    s_src = (ss0, ss1)
    s_dst = (sd0, sd1)
    s_w = (sw0, sw1)
    s_g = (sg0, sg1)
    s_sc = (sc0, sc1)

    def src_start(k, sl):
        pltpu.async_copy(src_hbm.at[wid, k], sbuf[sl], s_src[sl])

    def src_wait(sl):
        pltpu.make_async_copy(src_hbm.at[0, 0], sbuf[sl], s_src[sl]).wait()

    def dst_start(k, sl):
        pltpu.async_copy(dst_hbm.at[wid, k], dbuf[sl], s_dst[sl])

    def dst_wait(sl):
        pltpu.make_async_copy(dst_hbm.at[0, 0], dbuf[sl], s_dst[sl]).wait()

    def w_start(k, sl):
        pltpu.async_copy(w_hbm.at[wid, k], wbuf[sl], s_w[sl])

    def w_wait(sl):
        pltpu.make_async_copy(w_hbm.at[0, 0], wbuf[sl], s_w[sl]).wait()

    def gather_start(sl):
        pltpu.async_copy(h_hbm.at[sbuf[sl].at[0]], rows[sl], s_g[sl])

    def gather_wait(sl):
        pltpu.make_async_copy(h_hbm.at[sbuf[sl].at[0]], rows[sl],
                              s_g[sl]).wait()

    def scatter_start(sl):
        pltpu.async_copy(rows[sl], accum.at[dbuf[sl].at[0]], s_sc[sl],
                         add=True)

    def scatter_wait(sl):
        pltpu.make_async_copy(rows[sl], accum.at[dbuf[sl].at[0]],
                              s_sc[sl]).wait()

    def compute(sl):
        def grp(g, carry2):
            w16 = wbuf[sl][0, pl.ds(g * 16, 16)]
            for r in range(16):
                i = g * 16 + r
                wv = jnp.broadcast_to(w16[r], (16,))
                for j in range(HP // 16):
                    slc = pl.ds(j * 16, 16)
                    rows[sl][i, slc] = rows[sl][i, slc] * wv
            return carry2
        lax.fori_loop(0, C // 16, grp, 0)

    def process(k, sl, first, last):
        w_wait(sl)
        gather_wait(sl)
        @pl.when(k + 2 < NCHUNK)
        def _():
            src_start(k + 2, sl)
        compute(sl)
        @pl.when(k + 2 < NCHUNK)
        def _():
            w_start(k + 2, sl)
        if not first:
            scatter_wait(1 - sl)
        @pl.when(k + 1 < NCHUNK)
        def _():
            dst_start(k + 1, 1 - sl)
        dst_wait(sl)
        scatter_start(sl)
        if not last:
            src_wait(1 - sl)
            gather_start(1 - sl)

    src_start(0, 0)
    w_start(0, 0)
    dst_start(0, 0)
    src_start(1, 1)
    w_start(1, 1)
    src_wait(0)
    gather_start(0)

    process(0, 0, True, False)

    def body(j, carry):
        k = 2 * j + 1
        process(k, 1, False, False)
        process(k + 1, 0, False, False)
        return carry

    lax.fori_loop(0, (NCHUNK - 2) // 2, body, 0)
    process(NCHUNK - 2, 1, False, False)
    process(NCHUNK - 1, 0, False, True)
    scatter_wait(0)
    plsc.subcore_barrier()

    pltpu.sync_copy(accum.at[pl.ds(s * RPT, RPT)],
                    out_hbm.at[c, pl.ds(s * RPT, RPT)])


_seg = functools.partial(
    pl.kernel,
    out_type=jax.ShapeDtypeStruct((NC, NPAD, HP), jnp.float32),
    mesh=plsc.VectorSubcoreMesh(core_axis_name="c", subcore_axis_name="s"),
    scratch_types=[
        pltpu.VMEM_SHARED((NPAD, HP), jnp.float32),
        pltpu.VMEM((1, C), jnp.int32),
        pltpu.VMEM((1, C), jnp.int32),
        pltpu.VMEM((1, C), jnp.int32),
        pltpu.VMEM((1, C), jnp.int32),
        pltpu.VMEM((1, C), jnp.float32),
        pltpu.VMEM((1, C), jnp.float32),
        pltpu.VMEM((C, HP), jnp.float32),
        pltpu.VMEM((C, HP), jnp.float32),
        pltpu.SemaphoreType.DMA,
        pltpu.SemaphoreType.DMA,
        pltpu.SemaphoreType.DMA,
        pltpu.SemaphoreType.DMA,
        pltpu.SemaphoreType.DMA,
        pltpu.SemaphoreType.DMA,
        pltpu.SemaphoreType.DMA,
        pltpu.SemaphoreType.DMA,
        pltpu.SemaphoreType.DMA,
        pltpu.SemaphoreType.DMA,
    ],
)(_seg_body)


def _pad_to(a, rows, cols):
    return jnp.pad(a, ((0, rows - a.shape[0]), (0, cols - a.shape[1])))


def kernel(x, edge_index, edge_weight, W_lift, b_lift, W1, b1, W2, b2, W3, b3,
           W_out, b_out):
    src_t = edge_index[0].reshape(NW, NCHUNK, 1, C)
    dst_t = edge_index[1].reshape(NW, NCHUNK, 1, C)
    w_t = edge_weight.reshape(NW, NCHUNK, 1, C)
    zeros = jnp.zeros((RPT, HP), jnp.float32)

    wl_t = _pad_to(W_lift.T, D, HP)
    bl = jnp.pad(b_lift, (0, HP - H))[None, :]
    mats = []
    for W, b in ((W1, b1), (W2, b2), (W3, b3)):
        wat = _pad_to(W[:, :H].T, HP, HP)
        wbt = _pad_to(W[:, H:].T, HP, HP)
        mats.append((wat, wbt, jnp.pad(b, (0, HP - H))[None, :]))
    wo_t = _pad_to(W_out.T, HP, D)
    bo = b_out[None, :]

    h = _dense2(x, wl_t, bl, jnp.tanh)
    for wat, wbt, b in mats:
        r = _seg(h, src_t, dst_t, w_t, zeros)
        h = _update(h, r, wat, wbt, b)
    out = _dense2(h, wo_t, bo, jax.nn.sigmoid)
    return out


# speedup vs baseline: 8.5309x; 1.2446x over previous
import functools

import jax
import jax.numpy as jnp
from jax import lax
from jax.experimental import pallas as pl
from jax.experimental.pallas import tpu as pltpu
from jax.experimental.pallas import tpu_sc as plsc

N = 10000
E = 320000
D = 128
H = 100
HP = 128
BM = 2000

NC = 2
NS = 16
NW = NC * NS
EPW = E // NW
C = 80
NCHUNK = EPW // C
NPAD = 10240
WB = 632
WBL = N - (NS - 1) * WB


def _dense2_body(act, x_ref, w_ref, b_ref, o_ref):
    o_ref[...] = act(
        jnp.dot(x_ref[...], w_ref[...], preferred_element_type=jnp.float32)
        + b_ref[...]
    )


def _dense2(x, wt, b, act):
    m, k = x.shape
    f = wt.shape[1]
    return pl.pallas_call(
        functools.partial(_dense2_body, act),
        grid=(m // BM,),
        in_specs=[
            pl.BlockSpec((BM, k), lambda i: (i, 0)),
            pl.BlockSpec((k, f), lambda i: (0, 0)),
            pl.BlockSpec((1, f), lambda i: (0, 0)),
        ],
        out_specs=pl.BlockSpec((BM, f), lambda i: (i, 0)),
        out_shape=jax.ShapeDtypeStruct((m, f), jnp.float32),
    )(x, wt, b)


def _update_body(h_ref, r_ref, wa_ref, wb_ref, b_ref, o_ref):
    acc = jnp.dot(h_ref[...], wa_ref[...], preferred_element_type=jnp.float32)
    red = r_ref[0] + r_ref[1]
    acc += jnp.dot(red, wb_ref[...], preferred_element_type=jnp.float32)
    o_ref[...] = jnp.maximum(acc + b_ref[...], 0.0)


def _update(h, r, wat, wbt, b):
    return pl.pallas_call(
        _update_body,
        grid=(N // BM,),
        in_specs=[
            pl.BlockSpec((BM, HP), lambda i: (i, 0)),
            pl.BlockSpec((2, BM, HP), lambda i: (0, i, 0)),
            pl.BlockSpec((HP, HP), lambda i: (0, 0)),
            pl.BlockSpec((HP, HP), lambda i: (0, 0)),
            pl.BlockSpec((1, HP), lambda i: (0, 0)),
        ],
        out_specs=pl.BlockSpec((BM, HP), lambda i: (i, 0)),
        out_shape=jax.ShapeDtypeStruct((N, HP), jnp.float32),
    )(h, r, wat, wbt, b)


def _seg_body(h_hbm, src_hbm, dst_hbm, w_hbm, zeros_hbm, out_hbm,
              accum,
              sb0, sb1, sb2, db0, db1, db2, wb0, wb1, wb2, rv0, rv1, rv2,
              ss0, ss1, ss2, sd0, sd1, sd2, sw0, sw1, sw2,
              sg0, sg1, sg2, sc0, sc1, sc2):
    c = lax.axis_index("c")
    s = lax.axis_index("s")
    wid = c * NS + s

    @pl.when(s < NS - 1)
    def _():
        pltpu.sync_copy(zeros_hbm, accum.at[pl.ds(s * WB, WB)])

    @pl.when(s == NS - 1)
    def _():
        pltpu.sync_copy(zeros_hbm.at[pl.ds(0, WBL)],
                        accum.at[pl.ds((NS - 1) * WB, WBL)])
    plsc.subcore_barrier()

    sbuf = (sb0, sb1, sb2)
    dbuf = (db0, db1, db2)
    wbuf = (wb0, wb1, wb2)
    rows = (rv0, rv1, rv2)
    s_src = (ss0, ss1, ss2)
    s_dst = (sd0, sd1, sd2)
    s_w = (sw0, sw1, sw2)
    s_g = (sg0, sg1, sg2)
    s_sc = (sc0, sc1, sc2)

    def src_start(k, sl):
        pltpu.async_copy(src_hbm.at[wid, k], sbuf[sl], s_src[sl])

    def src_wait(sl):
        pltpu.make_async_copy(src_hbm.at[0, 0], sbuf[sl], s_src[sl]).wait()

    def dst_start(k, sl):
        pltpu.async_copy(dst_hbm.at[wid, k], dbuf[sl], s_dst[sl])

    def dst_wait(sl):
        pltpu.make_async_copy(dst_hbm.at[0, 0], dbuf[sl], s_dst[sl]).wait()

    def w_start(k, sl):
        pltpu.async_copy(w_hbm.at[wid, k], wbuf[sl], s_w[sl])

    def w_wait(sl):
        pltpu.make_async_copy(w_hbm.at[0, 0], wbuf[sl], s_w[sl]).wait()

    def gather_start(sl):
        pltpu.async_copy(h_hbm.at[sbuf[sl].at[0]], rows[sl], s_g[sl])

    def gather_wait(sl):
        pltpu.make_async_copy(h_hbm.at[sbuf[sl].at[0]], rows[sl],
                              s_g[sl]).wait()

    def scatter_start(sl):
        pltpu.async_copy(rows[sl], accum.at[dbuf[sl].at[0]], s_sc[sl],
                         add=True)

    def scatter_wait(sl):
        pltpu.make_async_copy(rows[sl], accum.at[dbuf[sl].at[0]],
                              s_sc[sl]).wait()

    def compute(sl):
        def grp(g, carry2):
            w16 = wbuf[sl][0, pl.ds(g * 16, 16)]
            for r in range(16):
                i = g * 16 + r
                wv = jnp.broadcast_to(w16[r], (16,))
                for j in range(HP // 16):
                    slc = pl.ds(j * 16, 16)
                    rows[sl][i, slc] = rows[sl][i, slc] * wv
            return carry2
        lax.fori_loop(0, C // 16, grp, 0)

    def process(k, sl):
        w_wait(sl)
        gather_wait(sl)
        sl1 = (sl + 1) % 3

        @pl.when(k + 3 < NCHUNK)
        def _():
            src_start(k + 3, sl)

        if not isinstance(k, int) or k >= 2:
            scatter_wait(sl1)
        @pl.when(k + 1 < NCHUNK)
        def _():
            src_wait(sl1)
            gather_start(sl1)
            dst_start(k + 1, sl1)
        compute(sl)

        @pl.when(k + 3 < NCHUNK)
        def _():
            w_start(k + 3, sl)
        dst_wait(sl)
        scatter_start(sl)

    src_start(0, 0)
    w_start(0, 0)
    dst_start(0, 0)
    src_start(1, 1)
    w_start(1, 1)
    src_start(2, 2)
    w_start(2, 2)
    src_wait(0)
    gather_start(0)

    process(0, 0)
    process(1, 1)

    def body(j, carry):
        k = 3 * j + 2
        process(k, 2)
        process(k + 1, 0)
        process(k + 2, 1)
        return carry

    lax.fori_loop(0, (NCHUNK - 2) // 3, body, 0)
    scatter_wait(0)
    scatter_wait(1)
    plsc.subcore_barrier()

    @pl.when(s < NS - 1)
    def _():
        pltpu.sync_copy(accum.at[pl.ds(s * WB, WB)],
                        out_hbm.at[c, pl.ds(s * WB, WB)])

    @pl.when(s == NS - 1)
    def _():
        pltpu.sync_copy(accum.at[pl.ds((NS - 1) * WB, WBL)],
                        out_hbm.at[c, pl.ds((NS - 1) * WB, WBL)])


_seg = functools.partial(
    pl.kernel,
    out_type=jax.ShapeDtypeStruct((NC, NPAD, HP), jnp.float32),
    mesh=plsc.VectorSubcoreMesh(core_axis_name="c", subcore_axis_name="s"),
    scratch_types=(
        [pltpu.VMEM_SHARED((N, HP), jnp.float32)]
        + [pltpu.VMEM((1, C), jnp.int32) for _ in range(6)]
        + [pltpu.VMEM((1, C), jnp.float32) for _ in range(3)]
        + [pltpu.VMEM((C, HP), jnp.float32) for _ in range(3)]
        + [pltpu.SemaphoreType.DMA for _ in range(15)]
    ),
)(_seg_body)


def _pad_to(a, rows, cols):
    return jnp.pad(a, ((0, rows - a.shape[0]), (0, cols - a.shape[1])))


def kernel(x, edge_index, edge_weight, W_lift, b_lift, W1, b1, W2, b2, W3, b3,
           W_out, b_out):
    src_t = edge_index[0].reshape(NW, NCHUNK, 1, C)
    dst_t = edge_index[1].reshape(NW, NCHUNK, 1, C)
    w_t = edge_weight.reshape(NW, NCHUNK, 1, C)
    zeros = jnp.zeros((WB, HP), jnp.float32)

    wl_t = _pad_to(W_lift.T, D, HP)
    bl = jnp.pad(b_lift, (0, HP - H))[None, :]
    mats = []
    for W, b in ((W1, b1), (W2, b2), (W3, b3)):
        wat = _pad_to(W[:, :H].T, HP, HP)
        wbt = _pad_to(W[:, H:].T, HP, HP)
        mats.append((wat, wbt, jnp.pad(b, (0, HP - H))[None, :]))
    wo_t = _pad_to(W_out.T, HP, D)
    bo = b_out[None, :]

    h = _dense2(x, wl_t, bl, jnp.tanh)
    for wat, wbt, b in mats:
        r = _seg(h, src_t, dst_t, w_t, zeros)
        h = _update(h, r, wat, wbt, b)
    out = _dense2(h, wo_t, bo, jax.nn.sigmoid)
    return out
